# baseline (speedup 1.0000x reference)
import functools

import jax
import jax.numpy as jnp
from jax import lax
from jax.experimental import pallas as pl
from jax.experimental.pallas import tpu as pltpu
from jax.experimental.pallas import tpu_sc as plsc

BATCH = 16384
EMB = 64
NC = 2
NS = 16
LANES = 16
NW = NC * NS
BPW = BATCH // NW
CHUNK = 128
NCH = BPW // CHUNK
GROUPS = BPW // LANES


def _sc_body(user_h, item_h, ubw_h, ibw_h, uew_h, iew_h, out_h,
             u_idx, i_idx, ue_v, ie_v, ub_v, ib_v, out_v, sem):
    wid = lax.axis_index("s") * NC + lax.axis_index("c")
    base = wid * BPW

    for c in range(NCH):
        pltpu.sync_copy(user_h.at[pl.ds(base + c * CHUNK, CHUNK)], u_idx.at[c])
        pltpu.sync_copy(item_h.at[pl.ds(base + c * CHUNK, CHUNK)], i_idx.at[c])

    copies = []
    for c in range(NCH):
        sl = pl.ds(c * CHUNK, CHUNK)
        copies.append(pltpu.async_copy(uew_h.at[u_idx.at[c]], ue_v.at[sl], sem))
        copies.append(pltpu.async_copy(iew_h.at[i_idx.at[c]], ie_v.at[sl], sem))
        copies.append(pltpu.async_copy(ubw_h.at[u_idx.at[c]], ub_v.at[sl], sem))
        copies.append(pltpu.async_copy(ibw_h.at[i_idx.at[c]], ib_v.at[sl], sem))
    for cp in copies:
        cp.wait()

    iota = lax.iota(jnp.int32, LANES)

    def group(g, _):
        base_r = g * LANES
        acc = ub_v[pl.ds(base_r, LANES)] + ib_v[pl.ds(base_r, LANES)]
        for l in range(LANES):
            b = base_r + l
            s = (ue_v[b, pl.ds(0, LANES)] * ie_v[b, pl.ds(0, LANES)])
            for k in range(1, EMB // LANES):
                s = s + (ue_v[b, pl.ds(k * LANES, LANES)]
                         * ie_v[b, pl.ds(k * LANES, LANES)])
            dot = jnp.sum(s)
            acc = acc + jnp.where(iota == l, dot, 0.0)
        out_v[pl.ds(base_r, LANES)] = acc
        return _

    lax.fori_loop(0, GROUPS, group, None)

    pltpu.sync_copy(out_v, out_h.at[pl.ds(base, BPW)])


@jax.jit
def _mf_biased_sc(user, item, ubw, ibw, uew, iew):
    mesh = plsc.VectorSubcoreMesh(core_axis_name="c", subcore_axis_name="s")
    return pl.kernel(
        _sc_body,
        out_type=jax.ShapeDtypeStruct((BATCH,), jnp.float32),
        mesh=mesh,
        compiler_params=pltpu.CompilerParams(needs_layout_passes=False,
                                             use_tc_tiling_on_sc=False),
        scratch_types=[
            pltpu.VMEM((NCH, CHUNK), jnp.int32),
            pltpu.VMEM((NCH, CHUNK), jnp.int32),
            pltpu.VMEM((BPW, EMB), jnp.float32),
            pltpu.VMEM((BPW, EMB), jnp.float32),
            pltpu.VMEM((BPW,), jnp.float32),
            pltpu.VMEM((BPW,), jnp.float32),
            pltpu.VMEM((BPW,), jnp.float32),
            pltpu.SemaphoreType.DMA,
        ],
    )(user, item, ubw, ibw, uew, iew)


def kernel(user, item, user_biases_w, item_biases_w, user_emb_w, item_emb_w):
    return _mf_biased_sc(user, item,
                         user_biases_w.reshape(-1), item_biases_w.reshape(-1),
                         user_emb_w, item_emb_w)
